# baseline (speedup 1.0000x reference)
import jax
import jax.numpy as jnp
from jax.experimental import pallas as pl
from jax.experimental.pallas import tpu as pltpu

B, T, D = 16, 2048, 128
H = 256
E = 128
LSTM = 128
NSH = 4
NEG = -1e30

T_BLK = 1024
TB = T // T_BLK
BT = B * T


def _encoder_kernel(len_ref, state_ref, len2d_ref, w1_ref, b1_ref, w2_ref,
                    b2_ref, wcat_ref, bg_ref, out_ref, emb_ref):
    b = pl.program_id(0)
    tb = pl.program_id(1)
    seq_len = len_ref[b]
    row0 = b * T + tb * T_BLK

    @pl.when(tb * T_BLK < seq_len)
    def _embed():
        x = state_ref[0]
        h = jnp.dot(x, w1_ref[:], preferred_element_type=jnp.float32) + b1_ref[:]
        h = jnp.maximum(h, 0.0)
        e = jnp.dot(h, w2_ref[:], preferred_element_type=jnp.float32) + b2_ref[:]
        emb_ref[pl.ds(row0, T_BLK), :] = e

    @pl.when(tb * T_BLK >= seq_len)
    def _zero():
        emb_ref[pl.ds(row0, T_BLK), :] = jnp.zeros((T_BLK, E), jnp.float32)

    @pl.when(jnp.logical_and(b == B - 1, tb == TB - 1))
    def _pool():
        emb = emb_ref[:]
        j = jax.lax.broadcasted_iota(jnp.int32, (B, BT), 1)
        base = jax.lax.broadcasted_iota(jnp.int32, (B, BT), 0) * T
        t_rel = j - base
        valid = jnp.logical_and(t_rel >= 0, t_rel < len2d_ref[:])
        addend = jnp.where(valid, 0.0, NEG)
        qt = jnp.zeros((B, LSTM), jnp.float32)
        ct = jnp.zeros((B, LSTM), jnp.float32)
        attended = jnp.zeros((B, E), jnp.float32)
        for _ in range(NSH):
            logit = jax.lax.dot_general(
                qt, emb, (((1,), (1,)), ((), ())),
                preferred_element_type=jnp.float32) + addend
            m = jnp.max(logit, axis=1, keepdims=True)
            w = jnp.exp(logit - m)
            s = jnp.sum(w, axis=1, keepdims=True)
            attended = jax.lax.dot_general(
                w, emb, (((1,), (0,)), ((), ())),
                preferred_element_type=jnp.float32) / s
            gates = jnp.dot(jnp.concatenate([attended, qt], axis=1),
                            wcat_ref[:],
                            preferred_element_type=jnp.float32) + bg_ref[:]
            i_g = jax.nn.sigmoid(gates[:, :LSTM])
            f_g = jax.nn.sigmoid(gates[:, LSTM:2 * LSTM])
            g_g = jnp.tanh(gates[:, 2 * LSTM:3 * LSTM])
            o_g = jax.nn.sigmoid(gates[:, 3 * LSTM:])
            ct = f_g * ct + i_g * g_g
            qt = o_g * jnp.tanh(ct)
        out_ref[:, :E] = attended
        out_ref[:, E:] = qt


def _state_imap(b, tb, len_ref):
    last = jax.lax.div(len_ref[b] + T_BLK - 1, T_BLK) - 1
    return (b, jnp.minimum(tb, last), 0)


def _full(b, tb, len_ref):
    return (0, 0)


def kernel(state, length, W1, b1, W2, b2, W_ih, W_hh, b_ih, b_hh):
    length = length.astype(jnp.int32)
    len2d = length.reshape(B, 1)
    wcat = jnp.concatenate([W_ih.T, W_hh.T], axis=0)
    grid_spec = pltpu.PrefetchScalarGridSpec(
        num_scalar_prefetch=1,
        grid=(B, TB),
        in_specs=[
            pl.BlockSpec((1, T_BLK, D), _state_imap),
            pl.BlockSpec((B, 1), _full),
            pl.BlockSpec((D, H), _full),
            pl.BlockSpec((1, H), _full),
            pl.BlockSpec((H, E), _full),
            pl.BlockSpec((1, E), _full),
            pl.BlockSpec((E + LSTM, 4 * LSTM), _full),
            pl.BlockSpec((1, 4 * LSTM), _full),
        ],
        out_specs=pl.BlockSpec((B, E + LSTM), _full),
        scratch_shapes=[pltpu.VMEM((BT, E), jnp.float32)],
    )
    return pl.pallas_call(
        _encoder_kernel,
        grid_spec=grid_spec,
        out_shape=jax.ShapeDtypeStruct((B, E + LSTM), jnp.float32),
        compiler_params=pltpu.CompilerParams(
            dimension_semantics=("arbitrary", "arbitrary")),
    )(length, state, len2d, W1.T, b1.reshape(1, H), W2.T, b2.reshape(1, E),
      wcat, (b_ih + b_hh).reshape(1, 4 * LSTM))


# speedup vs baseline: 1.0413x; 1.0413x over previous
import jax
import jax.numpy as jnp
from jax.experimental import pallas as pl
from jax.experimental.pallas import tpu as pltpu

B, T, D = 16, 2048, 128
H = 256
E = 128
LSTM = 128
NSH = 4
NEG = -1e30

T_BLK = 1024
TB = T // T_BLK
BT = B * T


def _encoder_kernel(len_ref, state_ref, len2d_ref, w1_ref, b1_ref, w2_ref,
                    b2_ref, wcat_ref, bg_ref, out_ref, emb_ref):
    b = pl.program_id(0)
    tb = pl.program_id(1)
    seq_len = len_ref[b]
    row0 = b * T + tb * T_BLK

    @pl.when(tb * T_BLK < seq_len)
    def _embed():
        x = state_ref[0].astype(jnp.bfloat16)
        h = jnp.dot(x, w1_ref[:], preferred_element_type=jnp.float32) + b1_ref[:]
        h = jnp.maximum(h, 0.0).astype(jnp.bfloat16)
        e = jnp.dot(h, w2_ref[:], preferred_element_type=jnp.float32) + b2_ref[:]
        emb_ref[pl.ds(row0, T_BLK), :] = e.astype(jnp.bfloat16)

    @pl.when(tb * T_BLK >= seq_len)
    def _zero():
        emb_ref[pl.ds(row0, T_BLK), :] = jnp.zeros((T_BLK, E), jnp.bfloat16)

    @pl.when(jnp.logical_and(b == B - 1, tb == TB - 1))
    def _pool():
        emb = emb_ref[:]
        j = jax.lax.broadcasted_iota(jnp.int32, (B, BT), 1)
        base = jax.lax.broadcasted_iota(jnp.int32, (B, BT), 0) * T
        t_rel = j - base
        valid = jnp.logical_and(t_rel >= 0, t_rel < len2d_ref[:])
        addend = jnp.where(valid, 0.0, NEG)
        len_f = len2d_ref[:].astype(jnp.float32)
        ct = jnp.zeros((B, LSTM), jnp.float32)
        qt = jnp.zeros((B, LSTM), jnp.float32)
        attended = jax.lax.dot_general(
            valid.astype(jnp.bfloat16), emb, (((1,), (0,)), ((), ())),
            preferred_element_type=jnp.float32) / len_f
        for it in range(NSH):
            gates = jnp.dot(
                jnp.concatenate([attended, qt], axis=1).astype(jnp.bfloat16),
                wcat_ref[:], preferred_element_type=jnp.float32) + bg_ref[:]
            i_g = jax.nn.sigmoid(gates[:, :LSTM])
            f_g = jax.nn.sigmoid(gates[:, LSTM:2 * LSTM])
            g_g = jnp.tanh(gates[:, 2 * LSTM:3 * LSTM])
            o_g = jax.nn.sigmoid(gates[:, 3 * LSTM:])
            ct = f_g * ct + i_g * g_g
            qt = o_g * jnp.tanh(ct)
            if it == NSH - 1:
                break
            logit = jax.lax.dot_general(
                qt.astype(jnp.bfloat16), emb, (((1,), (1,)), ((), ())),
                preferred_element_type=jnp.float32) + addend
            m = jnp.max(logit, axis=1, keepdims=True)
            w = jnp.exp(logit - m)
            s = jnp.sum(w, axis=1, keepdims=True)
            attended = jax.lax.dot_general(
                w.astype(jnp.bfloat16), emb, (((1,), (0,)), ((), ())),
                preferred_element_type=jnp.float32) / s
        out_ref[:, :E] = attended
        out_ref[:, E:] = qt


def _state_imap(b, tb, len_ref):
    last = jax.lax.div(len_ref[b] + T_BLK - 1, T_BLK) - 1
    return (b, jnp.minimum(tb, last), 0)


def _full(b, tb, len_ref):
    return (0, 0)


def kernel(state, length, W1, b1, W2, b2, W_ih, W_hh, b_ih, b_hh):
    length = length.astype(jnp.int32)
    len2d = length.reshape(B, 1)
    wcat = jnp.concatenate([W_ih.T, W_hh.T], axis=0)
    grid_spec = pltpu.PrefetchScalarGridSpec(
        num_scalar_prefetch=1,
        grid=(B, TB),
        in_specs=[
            pl.BlockSpec((1, T_BLK, D), _state_imap),
            pl.BlockSpec((B, 1), _full),
            pl.BlockSpec((D, H), _full),
            pl.BlockSpec((1, H), _full),
            pl.BlockSpec((H, E), _full),
            pl.BlockSpec((1, E), _full),
            pl.BlockSpec((E + LSTM, 4 * LSTM), _full),
            pl.BlockSpec((1, 4 * LSTM), _full),
        ],
        out_specs=pl.BlockSpec((B, E + LSTM), _full),
        scratch_shapes=[pltpu.VMEM((BT, E), jnp.bfloat16)],
    )
    return pl.pallas_call(
        _encoder_kernel,
        grid_spec=grid_spec,
        out_shape=jax.ShapeDtypeStruct((B, E + LSTM), jnp.float32),
        compiler_params=pltpu.CompilerParams(
            dimension_semantics=("arbitrary", "arbitrary")),
    )(length, state, len2d, W1.T.astype(jnp.bfloat16), b1.reshape(1, H),
      W2.T.astype(jnp.bfloat16), b2.reshape(1, E),
      wcat.astype(jnp.bfloat16), (b_ih + b_hh).reshape(1, 4 * LSTM))


# speedup vs baseline: 1.1118x; 1.0677x over previous
import jax
import jax.numpy as jnp
from jax.experimental import pallas as pl
from jax.experimental.pallas import tpu as pltpu

B, T, D = 16, 2048, 128
H = 256
E = 128
LSTM = 128
NSH = 4
NEG = -1e30

T_BLK = 1024
TB = T // T_BLK
NBLK = B * TB
BT = B * T


def _encoder_kernel(state_ref, len_ref, len2d_ref, w1_ref, b1_ref, w2_ref,
                    b2_ref, wcat_ref, bg_ref, out_ref, xbuf, emb_ref, sem):

    def block_valid(i):
        b, tb = divmod(i, TB)
        if tb == 0:
            return None
        return tb * T_BLK < len_ref[b]

    def copy(i):
        b, tb = divmod(i, TB)
        return pltpu.make_async_copy(
            state_ref.at[b, pl.ds(tb * T_BLK, T_BLK), :],
            xbuf.at[i % 2], sem.at[i % 2])

    def start(i):
        v = block_valid(i)
        if v is None:
            copy(i).start()
        else:
            @pl.when(v)
            def _():
                copy(i).start()

    def finish(i):
        b, tb = divmod(i, TB)
        row0 = b * T + tb * T_BLK
        v = block_valid(i)

        def _compute():
            copy(i).wait()
            x = xbuf[i % 2].astype(jnp.bfloat16)
            h = jnp.dot(x, w1_ref[:],
                        preferred_element_type=jnp.float32) + b1_ref[:]
            h = jnp.maximum(h, 0.0).astype(jnp.bfloat16)
            e = jnp.dot(h, w2_ref[:],
                        preferred_element_type=jnp.float32) + b2_ref[:]
            emb_ref[pl.ds(row0, T_BLK), :] = e.astype(jnp.bfloat16)

        if v is None:
            _compute()
        else:
            pl.when(v)(_compute)

            @pl.when(jnp.logical_not(v))
            def _zero():
                emb_ref[pl.ds(row0, T_BLK), :] = jnp.zeros((T_BLK, E),
                                                           jnp.bfloat16)

    start(0)
    for i in range(NBLK):
        if i + 1 < NBLK:
            start(i + 1)
        finish(i)

    emb = emb_ref[:]
    j = jax.lax.broadcasted_iota(jnp.int32, (B, BT), 1)
    base = jax.lax.broadcasted_iota(jnp.int32, (B, BT), 0) * T
    t_rel = j - base
    valid = jnp.logical_and(t_rel >= 0, t_rel < len2d_ref[:])
    addend = jnp.where(valid, 0.0, NEG)
    len_f = len2d_ref[:].astype(jnp.float32)
    ct = jnp.zeros((B, LSTM), jnp.float32)
    qt = jnp.zeros((B, LSTM), jnp.float32)
    attended = jax.lax.dot_general(
        valid.astype(jnp.bfloat16), emb, (((1,), (0,)), ((), ())),
        preferred_element_type=jnp.float32) / len_f
    for it in range(NSH):
        gates = jnp.dot(
            jnp.concatenate([attended, qt], axis=1).astype(jnp.bfloat16),
            wcat_ref[:], preferred_element_type=jnp.float32) + bg_ref[:]
        i_g = jax.nn.sigmoid(gates[:, :LSTM])
        f_g = jax.nn.sigmoid(gates[:, LSTM:2 * LSTM])
        g_g = jnp.tanh(gates[:, 2 * LSTM:3 * LSTM])
        o_g = jax.nn.sigmoid(gates[:, 3 * LSTM:])
        ct = f_g * ct + i_g * g_g
        qt = o_g * jnp.tanh(ct)
        if it == NSH - 1:
            break
        logit = jax.lax.dot_general(
            qt.astype(jnp.bfloat16), emb, (((1,), (1,)), ((), ())),
            preferred_element_type=jnp.float32) + addend
        m = jnp.max(logit, axis=1, keepdims=True)
        w = jnp.exp(logit - m)
        s = jnp.sum(w, axis=1, keepdims=True)
        attended = jax.lax.dot_general(
            w.astype(jnp.bfloat16), emb, (((1,), (0,)), ((), ())),
            preferred_element_type=jnp.float32) / s
    out_ref[:, :E] = attended
    out_ref[:, E:] = qt


def kernel(state, length, W1, b1, W2, b2, W_ih, W_hh, b_ih, b_hh):
    length = length.astype(jnp.int32)
    len2d = length.reshape(B, 1)
    wcat = jnp.concatenate([W_ih.T, W_hh.T], axis=0)
    return pl.pallas_call(
        _encoder_kernel,
        in_specs=[
            pl.BlockSpec(memory_space=pl.ANY),
            pl.BlockSpec(memory_space=pltpu.SMEM),
            pl.BlockSpec(memory_space=pltpu.VMEM),
            pl.BlockSpec(memory_space=pltpu.VMEM),
            pl.BlockSpec(memory_space=pltpu.VMEM),
            pl.BlockSpec(memory_space=pltpu.VMEM),
            pl.BlockSpec(memory_space=pltpu.VMEM),
            pl.BlockSpec(memory_space=pltpu.VMEM),
            pl.BlockSpec(memory_space=pltpu.VMEM),
        ],
        out_specs=pl.BlockSpec(memory_space=pltpu.VMEM),
        out_shape=jax.ShapeDtypeStruct((B, E + LSTM), jnp.float32),
        scratch_shapes=[
            pltpu.VMEM((2, T_BLK, D), jnp.float32),
            pltpu.VMEM((BT, E), jnp.bfloat16),
            pltpu.SemaphoreType.DMA((2,)),
        ],
    )(state, length, len2d, W1.T.astype(jnp.bfloat16), b1.reshape(1, H),
      W2.T.astype(jnp.bfloat16), b2.reshape(1, E),
      wcat.astype(jnp.bfloat16), (b_ih + b_hh).reshape(1, 4 * LSTM))


# speedup vs baseline: 1.4908x; 1.3409x over previous
import jax
import jax.numpy as jnp
from jax.experimental import pallas as pl
from jax.experimental.pallas import tpu as pltpu

B, T, D = 16, 2048, 128
H = 256
E = 128
LSTM = 128
NSH = 4
NEG = -1e30

BB = 4
NSTEP = B // BB
BT = B * T


def _encoder_kernel(state_ref, len2d_ref, w1_ref, b1_ref, w2_ref,
                    b2_ref, wcat_ref, bg_ref, out_ref, emb_ref):
    i = pl.program_id(0)

    x = state_ref[:].reshape(BB * T, D).astype(jnp.bfloat16)
    h = jnp.dot(x, w1_ref[:], preferred_element_type=jnp.float32) + b1_ref[:]
    h = jnp.maximum(h, 0.0).astype(jnp.bfloat16)
    e = jnp.dot(h, w2_ref[:], preferred_element_type=jnp.float32) + b2_ref[:]
    emb_ref[pl.ds(i * BB * T, BB * T), :] = e.astype(jnp.bfloat16)

    @pl.when(i == NSTEP - 1)
    def _pool():
        emb = emb_ref[:]
        j = jax.lax.broadcasted_iota(jnp.int32, (B, BT), 1)
        base = jax.lax.broadcasted_iota(jnp.int32, (B, BT), 0) * T
        t_rel = j - base
        valid = jnp.logical_and(t_rel >= 0, t_rel < len2d_ref[:])
        addend = jnp.where(valid, 0.0, NEG)
        len_f = len2d_ref[:].astype(jnp.float32)
        ct = jnp.zeros((B, LSTM), jnp.float32)
        qt = jnp.zeros((B, LSTM), jnp.float32)
        attended = jax.lax.dot_general(
            valid.astype(jnp.bfloat16), emb, (((1,), (0,)), ((), ())),
            preferred_element_type=jnp.float32) / len_f
        for it in range(NSH):
            gates = jnp.dot(
                jnp.concatenate([attended, qt], axis=1).astype(jnp.bfloat16),
                wcat_ref[:], preferred_element_type=jnp.float32) + bg_ref[:]
            i_g = jax.nn.sigmoid(gates[:, :LSTM])
            f_g = jax.nn.sigmoid(gates[:, LSTM:2 * LSTM])
            g_g = jnp.tanh(gates[:, 2 * LSTM:3 * LSTM])
            o_g = jax.nn.sigmoid(gates[:, 3 * LSTM:])
            ct = f_g * ct + i_g * g_g
            qt = o_g * jnp.tanh(ct)
            if it == NSH - 1:
                break
            logit = jax.lax.dot_general(
                qt.astype(jnp.bfloat16), emb, (((1,), (1,)), ((), ())),
                preferred_element_type=jnp.float32) + addend
            m = jnp.max(logit, axis=1, keepdims=True)
            w = jnp.exp(logit - m)
            s = jnp.sum(w, axis=1, keepdims=True)
            attended = jax.lax.dot_general(
                w.astype(jnp.bfloat16), emb, (((1,), (0,)), ((), ())),
                preferred_element_type=jnp.float32) / s
        out_ref[:, :E] = attended
        out_ref[:, E:] = qt


def _state_imap(i):
    return (i, 0, 0)


def _full(i):
    return (0, 0)


def kernel(state, length, W1, b1, W2, b2, W_ih, W_hh, b_ih, b_hh):
    length = length.astype(jnp.int32)
    len2d = length.reshape(B, 1)
    wcat = jnp.concatenate([W_ih.T, W_hh.T], axis=0)
    return pl.pallas_call(
        _encoder_kernel,
        grid=(NSTEP,),
        in_specs=[
            pl.BlockSpec((BB, T, D), _state_imap),
            pl.BlockSpec((B, 1), _full),
            pl.BlockSpec((D, H), _full),
            pl.BlockSpec((1, H), _full),
            pl.BlockSpec((H, E), _full),
            pl.BlockSpec((1, E), _full),
            pl.BlockSpec((E + LSTM, 4 * LSTM), _full),
            pl.BlockSpec((1, 4 * LSTM), _full),
        ],
        out_specs=pl.BlockSpec((B, E + LSTM), _full),
        out_shape=jax.ShapeDtypeStruct((B, E + LSTM), jnp.float32),
        scratch_shapes=[pltpu.VMEM((BT, E), jnp.bfloat16)],
        compiler_params=pltpu.CompilerParams(
            dimension_semantics=("arbitrary",)),
    )(state, len2d, W1.T.astype(jnp.bfloat16), b1.reshape(1, H),
      W2.T.astype(jnp.bfloat16), b2.reshape(1, E),
      wcat.astype(jnp.bfloat16), (b_ih + b_hh).reshape(1, 4 * LSTM))
